# baseline (speedup 1.0000x reference)
import jax
import jax.numpy as jnp
from jax.experimental import pallas as pl


def _concat_body(x_ref, pe_ref, po_ref, o_ref):
    xb = x_ref[...]
    bb, lh, _ = xb.shape
    pe = jnp.broadcast_to(pe_ref[...][None], (bb, lh, 64))
    po = jnp.broadcast_to(po_ref[...][None], (bb, lh, 64))
    o_ref[...] = jnp.concatenate(
        [xb[:, :, :64], pe, xb[:, :, 64:], po], axis=-1)


def kernel(x, position_table):
    B, L, D = x.shape
    pos = position_table[:L]
    pe = pos[0::2]
    po = pos[1::2]
    x2 = x.reshape(B, L // 2, 2 * D)
    BB = 128
    LH = L // 2
    out = pl.pallas_call(
        _concat_body,
        grid=(B // BB,),
        in_specs=[
            pl.BlockSpec((BB, LH, 2 * D), lambda i: (i, 0, 0)),
            pl.BlockSpec((LH, D), lambda i: (0, 0)),
            pl.BlockSpec((LH, D), lambda i: (0, 0)),
        ],
        out_specs=pl.BlockSpec((BB, LH, 4 * D), lambda i: (i, 0, 0)),
        out_shape=jax.ShapeDtypeStruct((B, LH, 4 * D), x.dtype),
    )(x2, pe, po)
    return out.reshape(B, L, 2 * D)


# speedup vs baseline: 1.5668x; 1.5668x over previous
import jax
import jax.numpy as jnp
from jax.experimental import pallas as pl


def _body(x_ref, pos_ref, o_ref):
    bb, flat = x_ref.shape
    L = flat // 64
    x3 = x_ref[...].reshape(bb, L, 64)
    pos = jnp.broadcast_to(pos_ref[...][None], (bb, L, 64))
    o_ref[...] = jnp.concatenate([x3, pos], axis=-1).reshape(bb, L * 128)


def kernel(x, position_table):
    B, L, D = x.shape
    pos = position_table[:L]
    x2 = x.reshape(B, L * D)
    BB = 128
    out = pl.pallas_call(
        _body,
        grid=(B // BB,),
        in_specs=[
            pl.BlockSpec((BB, L * D), lambda i: (i, 0)),
            pl.BlockSpec((L, D), lambda i: (0, 0)),
        ],
        out_specs=pl.BlockSpec((BB, L * 2 * D), lambda i: (i, 0)),
        out_shape=jax.ShapeDtypeStruct((B, L * 2 * D), x.dtype),
    )(x2, pos)
    return out.reshape(B, L, 2 * D)


# speedup vs baseline: 1.6075x; 1.0259x over previous
import jax
import jax.numpy as jnp
from jax.experimental import pallas as pl


def _body(x_ref, tmpl_ref, o_ref):
    bb = x_ref.shape[0]
    nt = x_ref.shape[1] // 128
    xb = x_ref[...]
    tiles = []
    for t in range(nt):
        xt = xb[:, 128 * t:128 * (t + 1)]
        tiles.append(xt[:, :64])
        tiles.append(jnp.broadcast_to(
            tmpl_ref[:, 256 * t + 64:256 * t + 128], (bb, 64)))
        tiles.append(xt[:, 64:])
        tiles.append(jnp.broadcast_to(
            tmpl_ref[:, 256 * t + 192:256 * t + 256], (bb, 64)))
    o_ref[...] = jnp.concatenate(tiles, axis=1)


def kernel(x, position_table):
    B, L, D = x.shape
    pos = position_table[:L]
    tmpl = jnp.concatenate(
        [jnp.zeros((L, D), pos.dtype), pos], axis=-1).reshape(1, L * 2 * D)
    x2 = x.reshape(B, L * D)
    BB = 128
    out = pl.pallas_call(
        _body,
        grid=(B // BB,),
        in_specs=[
            pl.BlockSpec((BB, L * D), lambda i: (i, 0)),
            pl.BlockSpec((1, L * 2 * D), lambda i: (0, 0)),
        ],
        out_specs=pl.BlockSpec((BB, L * 2 * D), lambda i: (i, 0)),
        out_shape=jax.ShapeDtypeStruct((B, L * 2 * D), x.dtype),
    )(x2, tmpl)
    return out.reshape(B, L, 2 * D)
